# Initial kernel scaffold; baseline (speedup 1.0000x reference)
import functools

import jax
import jax.numpy as jnp
from jax import lax
from jax.experimental import pallas as pl
from jax.experimental.pallas import tpu as pltpu
from jax.experimental.pallas import tpu_sc as plsc

N_NODES = 10000
D = 128

NC = 2
NS = 16
NW = NC * NS

CHUNK = 128
ROWS_PER_TILE = 640
N_PAD = NS * ROWS_PER_TILE


def _sc_agg_body(compute_deg, feat, src_hbm, dst_hbm, z_agg, z_deg, ones_hbm,
                 agg0_out, agg1_out, deg_out, src_v, dst_v, rows_v, ones_v,
                 tmpdeg_v, agg_s, deg_s, sem):
  c = lax.axis_index("c")
  s = lax.axis_index("s")
  w = c * NS + s
  n_chunks = src_hbm.shape[1]

  pltpu.sync_copy(src_hbm.at[w], src_v)
  pltpu.sync_copy(dst_hbm.at[w], dst_v)
  pltpu.sync_copy(z_agg, agg_s.at[pl.ds(s * ROWS_PER_TILE, ROWS_PER_TILE)])
  if compute_deg:
    pltpu.sync_copy(z_deg, deg_s.at[pl.ds(s * ROWS_PER_TILE, ROWS_PER_TILE)])
    pltpu.sync_copy(ones_hbm, ones_v)
  plsc.subcore_barrier()

  def chunk_body(ci, carry):
    pltpu.async_copy(feat.at[src_v.at[ci]], rows_v, sem).wait()
    pltpu.sync_copy(rows_v, agg_s.at[dst_v.at[ci]], add=True)
    if compute_deg:
      pltpu.sync_copy(ones_v, deg_s.at[dst_v.at[ci]], add=True)
    return carry

  lax.fori_loop(0, n_chunks, chunk_body, 0)
  plsc.subcore_barrier()

  for blk in range(ROWS_PER_TILE // CHUNK):
    r0 = s * ROWS_PER_TILE + blk * CHUNK

    @pl.when(c == 0)
    def _():
      pltpu.sync_copy(agg_s.at[pl.ds(r0, CHUNK)], rows_v)
      pltpu.sync_copy(rows_v, agg0_out.at[pl.ds(r0, CHUNK)])

    @pl.when(c == 1)
    def _():
      pltpu.sync_copy(agg_s.at[pl.ds(r0, CHUNK)], rows_v)
      pltpu.sync_copy(rows_v, agg1_out.at[pl.ds(r0, CHUNK)])

  if compute_deg:
    pltpu.sync_copy(deg_s.at[pl.ds(s * ROWS_PER_TILE, ROWS_PER_TILE)], tmpdeg_v)
    pltpu.sync_copy(tmpdeg_v, deg_out.at[c, pl.ds(s * ROWS_PER_TILE, ROWS_PER_TILE)])


def _make_sc_agg(n_chunks, compute_deg):
  mesh = plsc.VectorSubcoreMesh(core_axis_name="c", subcore_axis_name="s")
  out_type = [
      jax.ShapeDtypeStruct((N_PAD, D), jnp.float32),
      jax.ShapeDtypeStruct((N_PAD, D), jnp.float32),
  ]
  if compute_deg:
    out_type.append(jax.ShapeDtypeStruct((NC, N_PAD, 16), jnp.float32))
  scratch = [
      pltpu.VMEM((n_chunks, CHUNK), jnp.int32),
      pltpu.VMEM((n_chunks, CHUNK), jnp.int32),
      pltpu.VMEM((CHUNK, D), jnp.float32),
      pltpu.VMEM((CHUNK, 16), jnp.float32),
      pltpu.VMEM((ROWS_PER_TILE, 16), jnp.float32),
      pltpu.VMEM_SHARED((N_PAD, D), jnp.float32),
      pltpu.VMEM_SHARED((N_PAD, 16), jnp.float32),
      pltpu.SemaphoreType.DMA,
  ]

  if compute_deg:
    def body(feat, src_hbm, dst_hbm, z_agg, z_deg, ones_hbm,
             agg0, agg1, deg, *scr):
      _sc_agg_body(True, feat, src_hbm, dst_hbm, z_agg, z_deg, ones_hbm,
                   agg0, agg1, deg, *scr)
  else:
    def body(feat, src_hbm, dst_hbm, z_agg, z_deg, ones_hbm,
             agg0, agg1, *scr):
      _sc_agg_body(False, feat, src_hbm, dst_hbm, z_agg, z_deg, ones_hbm,
                   agg0, agg1, None, *scr)

  return pl.kernel(body, out_type=out_type, mesh=mesh, scratch_types=scratch,
                   name="sc_agg_deg" if compute_deg else "sc_agg")


def _tc_dense_body(relu, a0, a1, d0, d1, xr, wl, wr, b, o):
  deg = d0[:, 0:1] + d1[:, 0:1]
  inv = 1.0 / jnp.maximum(deg, 1.0)
  mean = (a0[...] + a1[...]) * inv
  acc = (jnp.dot(mean, wl[...], preferred_element_type=jnp.float32)
         + jnp.dot(xr[...], wr[...], preferred_element_type=jnp.float32)
         + b[...])
  o[...] = jnp.maximum(acc, 0.0) if relu else acc


def _make_tc_dense(relu, bn=1000):
  grid = (N_NODES // bn,)
  return pl.pallas_call(
      functools.partial(_tc_dense_body, relu),
      grid=grid,
      in_specs=[
          pl.BlockSpec((bn, D), lambda i: (i, 0)),
          pl.BlockSpec((bn, D), lambda i: (i, 0)),
          pl.BlockSpec((bn, 16), lambda i: (i, 0)),
          pl.BlockSpec((bn, 16), lambda i: (i, 0)),
          pl.BlockSpec((bn, D), lambda i: (i, 0)),
          pl.BlockSpec((D, D), lambda i: (0, 0)),
          pl.BlockSpec((D, D), lambda i: (0, 0)),
          pl.BlockSpec((1, D), lambda i: (0, 0)),
      ],
      out_specs=pl.BlockSpec((bn, D), lambda i: (i, 0)),
      out_shape=jax.ShapeDtypeStruct((N_NODES, D), jnp.float32),
      name="tc_dense_relu" if relu else "tc_dense",
  )


def kernel(x, edge_index, W1_l, W1_r, b1, W2_l, W2_r, b2):
  e = edge_index.shape[1]
  per_tile = -(-e // (NW * CHUNK)) * CHUNK
  n_chunks = per_tile // CHUNK
  e_pad = NW * per_tile

  src = edge_index[0].astype(jnp.int32)
  dst = edge_index[1].astype(jnp.int32)
  pad = e_pad - e
  src = jnp.concatenate([src, jnp.zeros((pad,), jnp.int32)]).reshape(NW, n_chunks, CHUNK)
  dst = jnp.concatenate([dst, jnp.full((pad,), N_NODES, jnp.int32)]).reshape(NW, n_chunks, CHUNK)

  z_agg = jnp.zeros((ROWS_PER_TILE, D), jnp.float32)
  z_deg = jnp.zeros((ROWS_PER_TILE, 16), jnp.float32)
  ones = jnp.ones((CHUNK, 16), jnp.float32)

  sc_agg1 = _make_sc_agg(n_chunks, compute_deg=True)
  sc_agg2 = _make_sc_agg(n_chunks, compute_deg=False)
  tc1 = _make_tc_dense(relu=True)
  tc2 = _make_tc_dense(relu=False)

  a0, a1, deg = sc_agg1(x, src, dst, z_agg, z_deg, ones)
  h = tc1(a0, a1, deg[0], deg[1], x, W1_l, W1_r, b1.reshape(1, D))
  c0, c1 = sc_agg2(h, src, dst, z_agg, z_deg, ones)
  out = tc2(c0, c1, deg[0], deg[1], h, W2_l, W2_r, b2.reshape(1, D))
  return out


# speedup vs baseline: 2.4102x; 2.4102x over previous
import functools

import jax
import jax.numpy as jnp
from jax import lax
from jax.experimental import pallas as pl
from jax.experimental.pallas import tpu as pltpu
from jax.experimental.pallas import tpu_sc as plsc

N_NODES = 10000
D = 128

NC = 2
NS = 16
NW = NC * NS

CHUNK = 128
ROWS_PER_TILE = 640
N_PAD = NS * ROWS_PER_TILE


def _sc_agg_body(feat, src_hbm, dst_hbm, z_agg, agg_out,
                 src_v, dst_v, rows_v, agg_s, sem):
  c = lax.axis_index("c")
  s = lax.axis_index("s")
  w = c * NS + s
  n_chunks = src_hbm.shape[1]

  pltpu.sync_copy(src_hbm.at[w], src_v)
  pltpu.sync_copy(dst_hbm.at[w], dst_v)
  pltpu.sync_copy(z_agg, rows_v)
  for zb in range(ROWS_PER_TILE // CHUNK):
    pltpu.sync_copy(rows_v,
                    agg_s.at[pl.ds(s * ROWS_PER_TILE + zb * CHUNK, CHUNK)])
  plsc.subcore_barrier()

  def chunk_body(ci, carry):
    pltpu.async_copy(feat.at[src_v.at[ci]], rows_v, sem).wait()
    pltpu.sync_copy(rows_v, agg_s.at[dst_v.at[ci]], add=True)
    return carry

  lax.fori_loop(0, n_chunks, chunk_body, 0)
  plsc.subcore_barrier()

  for blk in range(ROWS_PER_TILE // CHUNK):
    r0 = s * ROWS_PER_TILE + blk * CHUNK
    pltpu.sync_copy(agg_s.at[pl.ds(r0, CHUNK)], rows_v)
    pltpu.sync_copy(rows_v, agg_out.at[c, pl.ds(r0, CHUNK)])


def _make_sc_agg(n_chunks):
  mesh = plsc.VectorSubcoreMesh(core_axis_name="c", subcore_axis_name="s")
  out_type = jax.ShapeDtypeStruct((NC, N_PAD, D), jnp.float32)
  scratch = [
      pltpu.VMEM((n_chunks, CHUNK), jnp.int32),
      pltpu.VMEM((n_chunks, CHUNK), jnp.int32),
      pltpu.VMEM((CHUNK, D), jnp.float32),
      pltpu.VMEM_SHARED((N_PAD, D), jnp.float32),
      pltpu.SemaphoreType.DMA,
  ]
  return pl.kernel(_sc_agg_body, out_type=out_type, mesh=mesh,
                   scratch_types=scratch, name="sc_agg")


def _tc_dense_body(relu, a0, a1, d0, d1, xr, wl, wr, b, o):
  deg = d0[:, 0:1] + d1[:, 0:1]
  inv = 1.0 / jnp.maximum(deg, 1.0)
  mean = (a0[...] + a1[...]) * inv
  acc = (jnp.dot(mean, wl[...], preferred_element_type=jnp.float32)
         + jnp.dot(xr[...], wr[...], preferred_element_type=jnp.float32)
         + b[...])
  o[...] = jnp.maximum(acc, 0.0) if relu else acc


def _make_tc_dense(relu, bn=1000):
  grid = (N_NODES // bn,)
  return pl.pallas_call(
      functools.partial(_tc_dense_body, relu),
      grid=grid,
      in_specs=[
          pl.BlockSpec((bn, D), lambda i: (i, 0)),
          pl.BlockSpec((bn, D), lambda i: (i, 0)),
          pl.BlockSpec((bn, D), lambda i: (i, 0)),
          pl.BlockSpec((bn, D), lambda i: (i, 0)),
          pl.BlockSpec((bn, D), lambda i: (i, 0)),
          pl.BlockSpec((D, D), lambda i: (0, 0)),
          pl.BlockSpec((D, D), lambda i: (0, 0)),
          pl.BlockSpec((1, D), lambda i: (0, 0)),
      ],
      out_specs=pl.BlockSpec((bn, D), lambda i: (i, 0)),
      out_shape=jax.ShapeDtypeStruct((N_NODES, D), jnp.float32),
      name="tc_dense_relu" if relu else "tc_dense",
  )


def kernel(x, edge_index, W1_l, W1_r, b1, W2_l, W2_r, b2):
  e = edge_index.shape[1]
  n_chunks = -(-e // (NW * CHUNK * 8)) * 8
  per_tile = n_chunks * CHUNK
  e_pad = NW * per_tile

  src = edge_index[0].astype(jnp.int32)
  dst = edge_index[1].astype(jnp.int32)
  pad = e_pad - e
  src = jnp.concatenate([src, jnp.zeros((pad,), jnp.int32)]).reshape(NW, n_chunks, CHUNK)
  dst = jnp.concatenate([dst, jnp.full((pad,), N_NODES, jnp.int32)]).reshape(NW, n_chunks, CHUNK)

  z_agg = jnp.zeros((CHUNK, D), jnp.float32)
  ones_feat = jnp.ones((N_NODES, D), jnp.float32)

  sc_agg = _make_sc_agg(n_chunks)
  tc1 = _make_tc_dense(relu=True)
  tc2 = _make_tc_dense(relu=False)

  dd = sc_agg(ones_feat, src, dst, z_agg)
  aa = sc_agg(x, src, dst, z_agg)
  h = tc1(aa[0], aa[1], dd[0], dd[1], x, W1_l, W1_r, b1.reshape(1, D))
  cc = sc_agg(h, src, dst, z_agg)
  out = tc2(cc[0], cc[1], dd[0], dd[1], h, W2_l, W2_r, b2.reshape(1, D))
  return out
